# Initial kernel scaffold; baseline (speedup 1.0000x reference)
import jax
import jax.numpy as jnp
from jax.experimental import pallas as pl


def kernel(xyz, feats, W1, b1, g1, be1, W2, b2, g2, be2, W3, b3, g3, be3):
    raise NotImplementedError("write your pallas kernel here")


# speedup vs baseline: 4.7796x; 4.7796x over previous
import functools

import jax
import jax.numpy as jnp
from jax import lax
from jax.experimental import pallas as pl
from jax.experimental.pallas import tpu as pltpu
from jax.experimental.pallas import tpu_sc as plsc

_K = 32
_NW = 32


def _topk_body(c_ref, xt_ref, out_ref, *, P, K):
    b = pl.program_id(0)
    c = c_ref[0]
    xt = xt_ref[0]
    mt = c.shape[0]
    d0 = c[:, 0:1] - xt[0:1, :]
    d1 = c[:, 1:2] - xt[1:2, :]
    d2 = c[:, 2:3] - xt[2:3, :]
    d = d0 * d0 + d1 * d1 + d2 * d2
    iota = lax.broadcasted_iota(jnp.int32, (mt, P), 1)

    kcol = lax.broadcasted_iota(jnp.int32, (mt, K), 1)

    def body(j, carry):
        pv, pi, acc = carry
        valid = (d > pv) | ((d == pv) & (iota > pi))
        nv = jnp.min(jnp.where(valid, d, jnp.inf), axis=1, keepdims=True)
        ni = jnp.min(jnp.where(valid & (d == nv), iota, P), axis=1,
                     keepdims=True).astype(jnp.int32)
        acc = jnp.where(kcol == j, ni, acc)
        return nv, ni, acc

    _, _, acc = lax.fori_loop(
        0, K, body,
        (jnp.full((mt, 1), -jnp.inf, jnp.float32),
         jnp.full((mt, 1), -1, jnp.int32),
         jnp.zeros((mt, K), jnp.int32)))
    out_ref[0] = acc + b * P


def _topk(c4, x4t, *, P, K, MT=256):
    B, M, _ = c4.shape
    return pl.pallas_call(
        functools.partial(_topk_body, P=P, K=K),
        grid=(B, M // MT),
        in_specs=[
            pl.BlockSpec((1, MT, 4), lambda b, m: (b, m, 0)),
            pl.BlockSpec((1, 4, P), lambda b, m: (b, 0, 0)),
        ],
        out_specs=pl.BlockSpec((1, MT, K), lambda b, m: (b, m, 0)),
        out_shape=jax.ShapeDtypeStruct((B, M, K), jnp.int32),
    )(c4, x4t)


def _sc_gather(table, idx3, *, D):
    NW, CH, L = idx3.shape
    n_rows = NW * CH * L
    mesh = plsc.VectorSubcoreMesh(core_axis_name="c", subcore_axis_name="s")

    @functools.partial(
        pl.kernel, mesh=mesh,
        out_type=jax.ShapeDtypeStruct((n_rows, D), jnp.float32),
        scratch_types=[
            pltpu.VMEM((CH, L), jnp.int32),
            pltpu.VMEM((L, D), jnp.float32),
            pltpu.SemaphoreType.DMA,
        ],
    )
    def gk(table_hbm, idx_hbm, out_hbm, idx_v, rows_v, sem):
        wid = lax.axis_index("s") * 2 + lax.axis_index("c")
        pltpu.sync_copy(idx_hbm.at[wid], idx_v)

        def body(j, carry):
            pltpu.async_copy(table_hbm.at[idx_v.at[j]], rows_v, sem).wait()
            pltpu.sync_copy(rows_v, out_hbm.at[pl.ds((wid * CH + j) * L, L)])
            return carry

        lax.fori_loop(0, CH, body, 0)

    return gk(table, idx3)


def _colsums(y):
    return jnp.sum(y, axis=0, keepdims=True), jnp.sum(y * y, axis=0,
                                                      keepdims=True)


def _bn_coefs(s, q, n, g, be):
    mean = s / n
    var = q / n - mean * mean
    scale = g * lax.rsqrt(var + 1e-5)
    return scale, be - mean * scale


def _mlp_body(g_hbm, c_ref, w1_ref, w2_ref, w3_ref,
              b1_ref, g1_ref, be1_ref, b2_ref, g2_ref, be2_ref,
              b3_ref, g3_ref, be3_ref, out_ref, xt, va, vb, sem,
              *, M, K, NT):
    b = pl.program_id(0)
    N = M * K
    TS = N // NT
    MC = M // NT
    c = c_ref[0]
    w1 = w1_ref[...]
    w2 = w2_ref[...]
    w3 = w3_ref[...]
    hi = lax.Precision.HIGHEST
    corr = jnp.dot(c, w1[0:4, :], precision=hi,
                   preferred_element_type=jnp.float32)
    nf = jnp.float32(N)

    s1 = jnp.zeros((1, 128), jnp.float32)
    q1 = jnp.zeros((1, 128), jnp.float32)
    for t in range(NT):
        pltpu.make_async_copy(
            g_hbm.at[b, pl.ds(t * TS, TS), :], xt, sem).start()
        pltpu.make_async_copy(
            g_hbm.at[b, pl.ds(t * TS, TS), :], xt, sem).wait()
        y = jnp.dot(xt[...], w1, precision=hi,
                    preferred_element_type=jnp.float32)
        y = y + b1_ref[...]
        y = (y.reshape(MC, K, 128) - corr[t * MC:(t + 1) * MC, None, :]
             ).reshape(TS, 128)
        va[pl.ds(t * TS, TS), :] = y
        s, q = _colsums(y)
        s1, q1 = s1 + s, q1 + q
    sc1, sh1 = _bn_coefs(s1, q1, nf, g1_ref[...], be1_ref[...])

    s2 = jnp.zeros((1, 128), jnp.float32)
    q2 = jnp.zeros((1, 128), jnp.float32)
    for t in range(NT):
        h = jax.nn.relu(va[pl.ds(t * TS, TS), :] * sc1 + sh1)
        y = jnp.dot(h, w2, precision=hi,
                    preferred_element_type=jnp.float32) + b2_ref[...]
        vb[pl.ds(t * TS, TS), :] = y
        s, q = _colsums(y)
        s2, q2 = s2 + s, q2 + q
    sc2, sh2 = _bn_coefs(s2, q2, nf, g2_ref[...], be2_ref[...])

    s3 = jnp.zeros((1, 256), jnp.float32)
    q3 = jnp.zeros((1, 256), jnp.float32)
    for t in range(NT):
        h = jax.nn.relu(vb[pl.ds(t * TS, TS), :] * sc2 + sh2)
        y = jnp.dot(h, w3, precision=hi,
                    preferred_element_type=jnp.float32) + b3_ref[...]
        s, q = _colsums(y)
        s3, q3 = s3 + s, q3 + q
        va[pl.ds(t * TS, TS), :] = y[:, 0:128]
        vb[pl.ds(t * TS, TS), :] = y[:, 128:256]
    sc3, sh3 = _bn_coefs(s3, q3, nf, g3_ref[...], be3_ref[...])

    for t in range(NT):
        y = jnp.concatenate(
            [va[pl.ds(t * TS, TS), :], vb[pl.ds(t * TS, TS), :]], axis=1)
        h = jax.nn.relu(y * sc3 + sh3)
        out_ref[0, t * MC:(t + 1) * MC, :] = jnp.max(
            h.reshape(MC, K, 256), axis=1)


def _mlp(g, c4, w1p, w2t, w3t, bs, *, M, K, NT=8):
    B = c4.shape[0]
    OC = w3t.shape[1]
    N = M * K
    full = lambda s: pl.BlockSpec(s, lambda b: (0,) * len(s))
    return pl.pallas_call(
        functools.partial(_mlp_body, M=M, K=K, NT=NT),
        grid=(B,),
        in_specs=[
            pl.BlockSpec(memory_space=pl.ANY),
            pl.BlockSpec((1, M, 4), lambda b: (b, 0, 0)),
            full(w1p.shape), full(w2t.shape), full(w3t.shape),
        ] + [full(x.shape) for x in bs],
        out_specs=pl.BlockSpec((1, M, OC), lambda b: (b, 0, 0)),
        out_shape=jax.ShapeDtypeStruct((B, M, OC), jnp.float32),
        scratch_shapes=[
            pltpu.VMEM((N // NT, g.shape[-1]), jnp.float32),
            pltpu.VMEM((N, 128), jnp.float32),
            pltpu.VMEM((N, 128), jnp.float32),
            pltpu.SemaphoreType.DMA,
        ],
    )(g, c4, w1p, w2t, w3t, *bs)


def kernel(xyz, feats, W1, b1, g1, be1, W2, b2, g2, be2, W3, b3, g3, be3):
    B, P, _ = xyz.shape
    C = feats.shape[1]
    M = max(1, P // 4)
    K = min(_K, P)
    IC = 3 + C
    ICP = 128

    idx_center = jnp.linspace(0.0, P - 1, M).astype(jnp.int32)
    centers = xyz[:, idx_center, :]
    c4 = jnp.pad(centers, ((0, 0), (0, 0), (0, 1)))
    x4t = jnp.pad(jnp.transpose(xyz, (0, 2, 1)),
                  ((0, 0), (0, 1), (0, 0)))

    knn = _topk(c4, x4t, P=P, K=K)

    tbl = jnp.concatenate([xyz, jnp.transpose(feats, (0, 2, 1))], axis=-1)
    tbl = jnp.pad(tbl, ((0, 0), (0, 0), (0, ICP - IC))).reshape(B * P, ICP)
    idx3 = knn.reshape(_NW, (B * M * K) // (_NW * 128), 128)
    g = _sc_gather(tbl, idx3, D=ICP).reshape(B, M * K, ICP)

    w1p = jnp.pad(W1.T, ((0, ICP - IC), (0, 0)))
    bs = [b1.reshape(1, -1), g1.reshape(1, -1), be1.reshape(1, -1),
          b2.reshape(1, -1), g2.reshape(1, -1), be2.reshape(1, -1),
          b3.reshape(1, -1), g3.reshape(1, -1), be3.reshape(1, -1)]
    out = _mlp(g, c4, w1p, W2.T, W3.T, bs, M=M, K=K)

    return centers, jnp.transpose(out, (0, 2, 1))


# speedup vs baseline: 5.5640x; 1.1641x over previous
import functools

import jax
import jax.numpy as jnp
from jax import lax
from jax.experimental import pallas as pl
from jax.experimental.pallas import tpu as pltpu
from jax.experimental.pallas import tpu_sc as plsc

_K = 32
_NW = 32


def _topk_body(c_ref, xt_ref, out_ref, d3_ref, pv_ref, pi_ref,
               *, P, K, RC, L):
    c = c_ref[0]
    xt = xt_ref[0]
    mt = c.shape[0]
    S = P // L
    x3 = xt.reshape(4, S, L)
    d0 = c[:, 0:1, None] - x3[0][None]
    d1 = c[:, 1:2, None] - x3[1][None]
    d2 = c[:, 2:3, None] - x3[2][None]
    d3_ref[...] = d0 * d0 + d1 * d1 + d2 * d2
    pv_ref[...] = jnp.full((RC, mt, S), jnp.inf, jnp.float32)
    li = lax.broadcasted_iota(jnp.int32, (mt, S, L), 2)
    sbase = lax.broadcasted_iota(jnp.int32, (mt, S), 1) * L
    kcol = lax.broadcasted_iota(jnp.int32, (mt, K), 1)
    inf = jnp.inf

    def cond(st):
        r, conv = st
        return jnp.logical_and(r < RC, jnp.logical_not(conv))

    def round_body(st):
        r, _ = st
        dm = d3_ref[...]
        mn = jnp.min(dm, axis=2)
        mi = jnp.min(jnp.where(dm == mn[:, :, None], li, L),
                     axis=2).astype(jnp.int32)
        dm = jnp.where(li == mi[:, :, None], inf, dm)
        d3_ref[...] = dm
        pv_ref[pl.ds(r, 1)] = mn[None]
        pi_ref[pl.ds(r, 1)] = (sbase + mi)[None]
        rmin = jnp.min(dm, axis=(1, 2))
        cnt = jnp.sum((pv_ref[...] < rmin[None, :, None]).astype(jnp.int32),
                      axis=(0, 2))
        return r + 1, jnp.all(cnt >= K)

    _, conv = lax.while_loop(cond, round_body,
                             (jnp.int32(0), jnp.bool_(False)))

    @pl.when(conv)
    def fast():
        pooli = pi_ref[...]

        def body(j, acc):
            dm = pv_ref[...]
            nv = jnp.min(dm, axis=(0, 2))
            ni = jnp.min(jnp.where(dm == nv[None, :, None], pooli, P),
                         axis=(0, 2)).astype(jnp.int32)
            msk = (dm == nv[None, :, None]) & (pooli == ni[None, :, None])
            pv_ref[...] = jnp.where(msk, inf, dm)
            return jnp.where(kcol == j, ni[:, None], acc)

        out_ref[0] = lax.fori_loop(0, K, body,
                                   jnp.zeros((mt, K), jnp.int32))

    @pl.when(jnp.logical_not(conv))
    def slow():
        d3_ref[...] = d0 * d0 + d1 * d1 + d2 * d2
        gidx = sbase[:, :, None] + li

        def body(j, carry):
            pv, pi, acc = carry
            d3 = d3_ref[...]
            valid = (d3 > pv) | ((d3 == pv) & (gidx > pi))
            nv = jnp.min(jnp.where(valid, d3, inf), axis=(1, 2))
            nvb = nv[:, None, None]
            ni = jnp.min(jnp.where(valid & (d3 == nvb), gidx, P),
                         axis=(1, 2)).astype(jnp.int32)
            acc = jnp.where(kcol == j, ni[:, None], acc)
            return nvb, ni[:, None, None], acc

        _, _, acc = lax.fori_loop(
            0, K, body,
            (jnp.full((mt, 1, 1), -inf, jnp.float32),
             jnp.full((mt, 1, 1), -1, jnp.int32),
             jnp.zeros((mt, K), jnp.int32)))
        out_ref[0] = acc


def _topk(c4, x4t, *, P, K, MT=256, RC=12, L=128):
    B, M, _ = c4.shape
    S = P // L
    return pl.pallas_call(
        functools.partial(_topk_body, P=P, K=K, RC=RC, L=L),
        grid=(B, M // MT),
        in_specs=[
            pl.BlockSpec((1, MT, 4), lambda b, m: (b, m, 0)),
            pl.BlockSpec((1, 4, P), lambda b, m: (b, 0, 0)),
        ],
        out_specs=pl.BlockSpec((1, MT, K), lambda b, m: (b, m, 0)),
        out_shape=jax.ShapeDtypeStruct((B, M, K), jnp.int32),
        scratch_shapes=[
            pltpu.VMEM((MT, S, L), jnp.float32),
            pltpu.VMEM((RC, MT, S), jnp.float32),
            pltpu.VMEM((RC, MT, S), jnp.int32),
        ],
    )(c4, x4t)


def _sc_gather(table, idx3, *, D):
    NW, CH, L = idx3.shape
    n_rows = NW * CH * L
    mesh = plsc.VectorSubcoreMesh(core_axis_name="c", subcore_axis_name="s")

    @functools.partial(
        pl.kernel, mesh=mesh,
        out_type=jax.ShapeDtypeStruct((n_rows, D), jnp.float32),
        scratch_types=[
            pltpu.VMEM((CH, L), jnp.int32),
            pltpu.VMEM((L, D), jnp.float32),
            pltpu.SemaphoreType.DMA,
        ],
    )
    def gk(table_hbm, idx_hbm, out_hbm, idx_v, rows_v, sem):
        wid = lax.axis_index("s") * 2 + lax.axis_index("c")
        pltpu.sync_copy(idx_hbm.at[wid], idx_v)

        def body(j, carry):
            pltpu.async_copy(table_hbm.at[idx_v.at[j]], rows_v, sem).wait()
            pltpu.sync_copy(rows_v, out_hbm.at[pl.ds((wid * CH + j) * L, L)])
            return carry

        lax.fori_loop(0, CH, body, 0)

    return gk(table, idx3)


def _colsums(y):
    return jnp.sum(y, axis=0, keepdims=True), jnp.sum(y * y, axis=0,
                                                      keepdims=True)


def _bn_coefs(s, q, n, g, be):
    mean = s / n
    var = q / n - mean * mean
    scale = g * lax.rsqrt(var + 1e-5)
    return scale, be - mean * scale


def _mlp_body(g_hbm, c_ref, w1_ref, w2_ref, w3_ref,
              b1_ref, g1_ref, be1_ref, b2_ref, g2_ref, be2_ref,
              b3_ref, g3_ref, be3_ref, out_ref, xt0, xt1, va, vb,
              sem0, sem1, *, M, K, NT):
    b = pl.program_id(0)
    N = M * K
    xts = (xt0, xt1)
    sems = (sem0, sem1)
    TS = N // NT
    MC = M // NT
    c = c_ref[0]
    w1 = w1_ref[...]
    w2 = w2_ref[...]
    w3 = w3_ref[...]
    hi = lax.Precision.HIGHEST
    corr = jnp.dot(c, w1[0:4, :], precision=hi,
                   preferred_element_type=jnp.float32)
    nf = jnp.float32(N)

    s1 = jnp.zeros((1, 128), jnp.float32)
    q1 = jnp.zeros((1, 128), jnp.float32)
    pltpu.make_async_copy(g_hbm.at[b, pl.ds(0, TS), :], xt0, sem0).start()
    for t in range(NT):
        if t + 1 < NT:
            pltpu.make_async_copy(
                g_hbm.at[b, pl.ds((t + 1) * TS, TS), :],
                xts[(t + 1) % 2], sems[(t + 1) % 2]).start()
        pltpu.make_async_copy(
            g_hbm.at[b, pl.ds(t * TS, TS), :], xts[t % 2],
            sems[t % 2]).wait()
        y = jnp.dot(xts[t % 2][...], w1, precision=hi,
                    preferred_element_type=jnp.float32)
        y = y + b1_ref[...]
        y = (y.reshape(MC, K, 128) - corr[t * MC:(t + 1) * MC, None, :]
             ).reshape(TS, 128)
        va[pl.ds(t * TS, TS), :] = y
        s, q = _colsums(y)
        s1, q1 = s1 + s, q1 + q
    sc1, sh1 = _bn_coefs(s1, q1, nf, g1_ref[...], be1_ref[...])

    s2 = jnp.zeros((1, 128), jnp.float32)
    q2 = jnp.zeros((1, 128), jnp.float32)
    for t in range(NT):
        h = jax.nn.relu(va[pl.ds(t * TS, TS), :] * sc1 + sh1)
        y = jnp.dot(h, w2, precision=hi,
                    preferred_element_type=jnp.float32) + b2_ref[...]
        vb[pl.ds(t * TS, TS), :] = y
        s, q = _colsums(y)
        s2, q2 = s2 + s, q2 + q
    sc2, sh2 = _bn_coefs(s2, q2, nf, g2_ref[...], be2_ref[...])

    s3 = jnp.zeros((1, 256), jnp.float32)
    q3 = jnp.zeros((1, 256), jnp.float32)
    for t in range(NT):
        h = jax.nn.relu(vb[pl.ds(t * TS, TS), :] * sc2 + sh2)
        y = jnp.dot(h, w3, precision=hi,
                    preferred_element_type=jnp.float32) + b3_ref[...]
        s, q = _colsums(y)
        s3, q3 = s3 + s, q3 + q
        va[pl.ds(t * TS, TS), :] = y[:, 0:128]
        vb[pl.ds(t * TS, TS), :] = y[:, 128:256]
    sc3, sh3 = _bn_coefs(s3, q3, nf, g3_ref[...], be3_ref[...])

    for t in range(NT):
        y = jnp.concatenate(
            [va[pl.ds(t * TS, TS), :], vb[pl.ds(t * TS, TS), :]], axis=1)
        h = jax.nn.relu(y * sc3 + sh3)
        out_ref[0, t * MC:(t + 1) * MC, :] = jnp.max(
            h.reshape(MC, K, 256), axis=1)


def _mlp(g, c4, w1p, w2t, w3t, bs, *, M, K, NT=8):
    B = c4.shape[0]
    OC = w3t.shape[1]
    N = M * K
    full = lambda s: pl.BlockSpec(s, lambda b: (0,) * len(s))
    return pl.pallas_call(
        functools.partial(_mlp_body, M=M, K=K, NT=NT),
        grid=(B,),
        in_specs=[
            pl.BlockSpec(memory_space=pl.ANY),
            pl.BlockSpec((1, M, 4), lambda b: (b, 0, 0)),
            full(w1p.shape), full(w2t.shape), full(w3t.shape),
        ] + [full(x.shape) for x in bs],
        out_specs=pl.BlockSpec((1, M, OC), lambda b: (b, 0, 0)),
        out_shape=jax.ShapeDtypeStruct((B, M, OC), jnp.float32),
        scratch_shapes=[
            pltpu.VMEM((N // NT, g.shape[-1]), jnp.float32),
            pltpu.VMEM((N // NT, g.shape[-1]), jnp.float32),
            pltpu.VMEM((N, 128), jnp.float32),
            pltpu.VMEM((N, 128), jnp.float32),
            pltpu.SemaphoreType.DMA,
            pltpu.SemaphoreType.DMA,
        ],
    )(g, c4, w1p, w2t, w3t, *bs)


def kernel(xyz, feats, W1, b1, g1, be1, W2, b2, g2, be2, W3, b3, g3, be3):
    B, P, _ = xyz.shape
    C = feats.shape[1]
    M = max(1, P // 4)
    K = min(_K, P)
    IC = 3 + C
    ICP = 128

    idx_center = jnp.linspace(0.0, P - 1, M).astype(jnp.int32)
    centers = xyz[:, idx_center, :]
    c4 = jnp.pad(centers, ((0, 0), (0, 0), (0, 1)))
    x4t = jnp.pad(jnp.transpose(xyz, (0, 2, 1)),
                  ((0, 0), (0, 1), (0, 0)))

    knn = _topk(c4, x4t, P=P, K=K)
    base = (jnp.arange(B, dtype=jnp.int32) * P).reshape(B, 1, 1)
    knng = knn + base

    tbl = jnp.concatenate([xyz, jnp.transpose(feats, (0, 2, 1))], axis=-1)
    tbl = jnp.pad(tbl, ((0, 0), (0, 0), (0, ICP - IC))).reshape(B * P, ICP)

    w1p = jnp.pad(W1.T, ((0, ICP - IC), (0, 0)))
    bs = [b1.reshape(1, -1), g1.reshape(1, -1), be1.reshape(1, -1),
          b2.reshape(1, -1), g2.reshape(1, -1), be2.reshape(1, -1),
          b3.reshape(1, -1), g3.reshape(1, -1), be3.reshape(1, -1)]

    outs = []
    for b in range(B):
        idx3 = knng[b].reshape(_NW, (M * K) // (_NW * 128), 128)
        gb = _sc_gather(tbl, idx3, D=ICP).reshape(1, M * K, ICP)
        outs.append(_mlp(gb, c4[b:b + 1], w1p, W2.T, W3.T, bs, M=M, K=K))
    out = jnp.concatenate(outs, axis=0)

    return centers, jnp.transpose(out, (0, 2, 1))


# speedup vs baseline: 5.7517x; 1.0337x over previous
import functools

import jax
import jax.numpy as jnp
from jax import lax
from jax.experimental import pallas as pl
from jax.experimental.pallas import tpu as pltpu
from jax.experimental.pallas import tpu_sc as plsc

_K = 32
_NW = 32


def _topk_body(c_ref, xt_ref, out_ref, d3_ref, pv_ref, pi_ref, done_ref,
               *, P, K, RC, L):
    c = c_ref[0]
    xt = xt_ref[0]
    mt = c.shape[0]
    S = P // L
    x3 = xt.reshape(4, S, L)
    d0 = c[:, 0:1, None] - x3[0][None]
    d1 = c[:, 1:2, None] - x3[1][None]
    d2 = c[:, 2:3, None] - x3[2][None]
    d3_ref[...] = d0 * d0 + d1 * d1 + d2 * d2
    pv_ref[...] = jnp.full((mt, RC * S), jnp.inf, jnp.float32)
    done_ref[0] = 0
    li = lax.broadcasted_iota(jnp.int32, (mt, S, L), 2)
    sbase = lax.broadcasted_iota(jnp.int32, (mt, S), 1) * L
    kcol = lax.broadcasted_iota(jnp.int32, (mt, K), 1)
    inf = jnp.inf

    for r in range(RC):
        @pl.when(done_ref[0] == 0)
        def _round(r=r):
            dm = d3_ref[...]
            mn = jnp.min(dm, axis=2)
            mi = jnp.min(jnp.where(dm == mn[:, :, None], li, L),
                         axis=2).astype(jnp.int32)
            d3_ref[...] = jnp.where(li == mi[:, :, None], inf, dm)
            pv_ref[:, r * S:(r + 1) * S] = mn
            pi_ref[:, r * S:(r + 1) * S] = sbase + mi
            rlb = jnp.min(mn, axis=1, keepdims=True)
            cnt = jnp.sum((pv_ref[...] < rlb).astype(jnp.int32), axis=1)
            done_ref[0] = jnp.all(cnt >= K).astype(jnp.int32)

    @pl.when(done_ref[0] == 1)
    def _fast():
        pooli = pi_ref[...]

        def body(j, acc):
            dm = pv_ref[...]
            nv = jnp.min(dm, axis=1)
            ni = jnp.min(jnp.where(dm == nv[:, None], pooli, P),
                         axis=1).astype(jnp.int32)
            msk = (dm == nv[:, None]) & (pooli == ni[:, None])
            pv_ref[...] = jnp.where(msk, inf, dm)
            return jnp.where(kcol == j, ni[:, None], acc)

        out_ref[0] = lax.fori_loop(0, K, body,
                                   jnp.zeros((mt, K), jnp.int32))

    @pl.when(done_ref[0] == 0)
    def _slow():
        d3_ref[...] = d0 * d0 + d1 * d1 + d2 * d2
        gidx = sbase[:, :, None] + li

        def body(j, carry):
            pv, pi, acc = carry
            d3 = d3_ref[...]
            valid = (d3 > pv) | ((d3 == pv) & (gidx > pi))
            nv = jnp.min(jnp.where(valid, d3, inf), axis=(1, 2))
            nvb = nv[:, None, None]
            ni = jnp.min(jnp.where(valid & (d3 == nvb), gidx, P),
                         axis=(1, 2)).astype(jnp.int32)
            acc = jnp.where(kcol == j, ni[:, None], acc)
            return nvb, ni[:, None, None], acc

        _, _, acc = lax.fori_loop(
            0, K, body,
            (jnp.full((mt, 1, 1), -inf, jnp.float32),
             jnp.full((mt, 1, 1), -1, jnp.int32),
             jnp.zeros((mt, K), jnp.int32)))
        out_ref[0] = acc


def _topk(c4, x4t, *, P, K, MT=256, RC=12, L=128):
    B, M, _ = c4.shape
    S = P // L
    return pl.pallas_call(
        functools.partial(_topk_body, P=P, K=K, RC=RC, L=L),
        grid=(B, M // MT),
        in_specs=[
            pl.BlockSpec((1, MT, 4), lambda b, m: (b, m, 0)),
            pl.BlockSpec((1, 4, P), lambda b, m: (b, 0, 0)),
        ],
        out_specs=pl.BlockSpec((1, MT, K), lambda b, m: (b, m, 0)),
        out_shape=jax.ShapeDtypeStruct((B, M, K), jnp.int32),
        scratch_shapes=[
            pltpu.VMEM((MT, S, L), jnp.float32),
            pltpu.VMEM((MT, RC * S), jnp.float32),
            pltpu.VMEM((MT, RC * S), jnp.int32),
            pltpu.SMEM((1,), jnp.int32),
        ],
    )(c4, x4t)


def _sc_gather(table, idx3, *, D):
    NW, CH, L = idx3.shape
    n_rows = NW * CH * L
    mesh = plsc.VectorSubcoreMesh(core_axis_name="c", subcore_axis_name="s")

    @functools.partial(
        pl.kernel, mesh=mesh,
        out_type=jax.ShapeDtypeStruct((n_rows, D), jnp.float32),
        scratch_types=[
            pltpu.VMEM((CH, L), jnp.int32),
            pltpu.VMEM((L, D), jnp.float32),
            pltpu.SemaphoreType.DMA,
        ],
    )
    def gk(table_hbm, idx_hbm, out_hbm, idx_v, rows_v, sem):
        wid = lax.axis_index("s") * 2 + lax.axis_index("c")
        pltpu.sync_copy(idx_hbm.at[wid], idx_v)

        def body(j, carry):
            pltpu.async_copy(table_hbm.at[idx_v.at[j]], rows_v, sem).wait()
            pltpu.sync_copy(rows_v, out_hbm.at[pl.ds((wid * CH + j) * L, L)])
            return carry

        lax.fori_loop(0, CH, body, 0)

    return gk(table, idx3)


def _colsums(y):
    return jnp.sum(y, axis=0, keepdims=True), jnp.sum(y * y, axis=0,
                                                      keepdims=True)


def _bn_coefs(s, q, n, g, be):
    mean = s / n
    var = q / n - mean * mean
    scale = g * lax.rsqrt(var + 1e-5)
    return scale, be - mean * scale


def _mlp_body(g_hbm, c_ref, w1_ref, w2_ref, w3_ref,
              b1_ref, g1_ref, be1_ref, b2_ref, g2_ref, be2_ref,
              b3_ref, g3_ref, be3_ref, out_ref, xt0, xt1, va, vb,
              sem0, sem1, *, M, K, NT):
    b = pl.program_id(0)
    N = M * K
    xts = (xt0, xt1)
    sems = (sem0, sem1)
    TS = N // NT
    MC = M // NT
    c = c_ref[0]
    w1 = w1_ref[...]
    w2 = w2_ref[...]
    w3 = w3_ref[...]
    hi = lax.Precision.HIGHEST
    corr = jnp.dot(c, w1[0:4, :], precision=hi,
                   preferred_element_type=jnp.float32)
    nf = jnp.float32(N)

    s1 = jnp.zeros((1, 128), jnp.float32)
    q1 = jnp.zeros((1, 128), jnp.float32)
    pltpu.make_async_copy(g_hbm.at[b, pl.ds(0, TS), :], xt0, sem0).start()
    for t in range(NT):
        if t + 1 < NT:
            pltpu.make_async_copy(
                g_hbm.at[b, pl.ds((t + 1) * TS, TS), :],
                xts[(t + 1) % 2], sems[(t + 1) % 2]).start()
        pltpu.make_async_copy(
            g_hbm.at[b, pl.ds(t * TS, TS), :], xts[t % 2],
            sems[t % 2]).wait()
        y = jnp.dot(xts[t % 2][...], w1, precision=hi,
                    preferred_element_type=jnp.float32)
        y = y + b1_ref[...]
        y = (y.reshape(MC, K, 128) - corr[t * MC:(t + 1) * MC, None, :]
             ).reshape(TS, 128)
        va[pl.ds(t * TS, TS), :] = y
        s, q = _colsums(y)
        s1, q1 = s1 + s, q1 + q
    sc1, sh1 = _bn_coefs(s1, q1, nf, g1_ref[...], be1_ref[...])

    s2 = jnp.zeros((1, 128), jnp.float32)
    q2 = jnp.zeros((1, 128), jnp.float32)
    for t in range(NT):
        h = jax.nn.relu(va[pl.ds(t * TS, TS), :] * sc1 + sh1)
        y = jnp.dot(h, w2, precision=hi,
                    preferred_element_type=jnp.float32) + b2_ref[...]
        vb[pl.ds(t * TS, TS), :] = y
        s, q = _colsums(y)
        s2, q2 = s2 + s, q2 + q
    sc2, sh2 = _bn_coefs(s2, q2, nf, g2_ref[...], be2_ref[...])

    s3 = jnp.zeros((1, 256), jnp.float32)
    q3 = jnp.zeros((1, 256), jnp.float32)
    for t in range(NT):
        h = jax.nn.relu(vb[pl.ds(t * TS, TS), :] * sc2 + sh2)
        y = jnp.dot(h, w3, precision=hi,
                    preferred_element_type=jnp.float32) + b3_ref[...]
        s, q = _colsums(y)
        s3, q3 = s3 + s, q3 + q
        va[pl.ds(t * TS, TS), :] = y[:, 0:128]
        vb[pl.ds(t * TS, TS), :] = y[:, 128:256]
    sc3, sh3 = _bn_coefs(s3, q3, nf, g3_ref[...], be3_ref[...])

    for t in range(NT):
        y = jnp.concatenate(
            [va[pl.ds(t * TS, TS), :], vb[pl.ds(t * TS, TS), :]], axis=1)
        h = jax.nn.relu(y * sc3 + sh3)
        out_ref[0, t * MC:(t + 1) * MC, :] = jnp.max(
            h.reshape(MC, K, 256), axis=1)


def _mlp(g, c4, w1p, w2t, w3t, bs, *, M, K, NT=8):
    B = c4.shape[0]
    OC = w3t.shape[1]
    N = M * K
    full = lambda s: pl.BlockSpec(s, lambda b: (0,) * len(s))
    return pl.pallas_call(
        functools.partial(_mlp_body, M=M, K=K, NT=NT),
        grid=(B,),
        in_specs=[
            pl.BlockSpec(memory_space=pl.ANY),
            pl.BlockSpec((1, M, 4), lambda b: (b, 0, 0)),
            full(w1p.shape), full(w2t.shape), full(w3t.shape),
        ] + [full(x.shape) for x in bs],
        out_specs=pl.BlockSpec((1, M, OC), lambda b: (b, 0, 0)),
        out_shape=jax.ShapeDtypeStruct((B, M, OC), jnp.float32),
        scratch_shapes=[
            pltpu.VMEM((N // NT, g.shape[-1]), jnp.float32),
            pltpu.VMEM((N // NT, g.shape[-1]), jnp.float32),
            pltpu.VMEM((N, 128), jnp.float32),
            pltpu.VMEM((N, 128), jnp.float32),
            pltpu.SemaphoreType.DMA,
            pltpu.SemaphoreType.DMA,
        ],
    )(g, c4, w1p, w2t, w3t, *bs)


def kernel(xyz, feats, W1, b1, g1, be1, W2, b2, g2, be2, W3, b3, g3, be3):
    B, P, _ = xyz.shape
    C = feats.shape[1]
    M = max(1, P // 4)
    K = min(_K, P)
    IC = 3 + C
    ICP = 128

    idx_center = jnp.linspace(0.0, P - 1, M).astype(jnp.int32)
    centers = xyz[:, idx_center, :]
    c4 = jnp.pad(centers, ((0, 0), (0, 0), (0, 1)))
    x4t = jnp.pad(jnp.transpose(xyz, (0, 2, 1)),
                  ((0, 0), (0, 1), (0, 0)))

    knn = _topk(c4, x4t, P=P, K=K)
    base = (jnp.arange(B, dtype=jnp.int32) * P).reshape(B, 1, 1)
    knng = knn + base

    tbl = jnp.concatenate([xyz, jnp.transpose(feats, (0, 2, 1))], axis=-1)
    tbl = jnp.pad(tbl, ((0, 0), (0, 0), (0, ICP - IC))).reshape(B * P, ICP)

    w1p = jnp.pad(W1.T, ((0, ICP - IC), (0, 0)))
    bs = [b1.reshape(1, -1), g1.reshape(1, -1), be1.reshape(1, -1),
          b2.reshape(1, -1), g2.reshape(1, -1), be2.reshape(1, -1),
          b3.reshape(1, -1), g3.reshape(1, -1), be3.reshape(1, -1)]

    outs = []
    for b in range(B):
        idx3 = knng[b].reshape(_NW, (M * K) // (_NW * 128), 128)
        gb = _sc_gather(tbl, idx3, D=ICP).reshape(1, M * K, ICP)
        outs.append(_mlp(gb, c4[b:b + 1], w1p, W2.T, W3.T, bs, M=M, K=K))
    out = jnp.concatenate(outs, axis=0)

    return centers, jnp.transpose(out, (0, 2, 1))
